# baseline (speedup 1.0000x reference)
import functools
import math

import jax
import jax.numpy as jnp
from jax.experimental import pallas as pl
from jax.experimental.pallas import tpu as pltpu

_GAMMA = 0.1
_BIG = 1e6
_C1 = -math.log2(math.e) / _GAMMA
_C2 = -_GAMMA * math.log(2.0)


def _sdtw_kernel(x_ref, y_ref, out_ref, *, N, M):
    x = x_ref[:, :]
    big = x * 0.0 + _BIG * _C1
    big_row = big[:1, :]
    zero_row = big_row * 0.0
    H = N // 2

    def make_body(xs, off):
        L = xs.shape[0]

        def finish(k, b0, b1, b2):
            yw = y_ref[pl.ds(off + N + M - k, L), :]
            d = jnp.abs(xs - yw)
            bmax = jnp.maximum(jnp.maximum(b0, b1), b2)
            rsum = jnp.exp2(b0 - bmax) + jnp.exp2(b1 - bmax) + jnp.exp2(b2 - bmax)
            return (jnp.log2(rsum) + bmax) - d

        def body(k, v_km2, v_km1, r0_row, r1_row):
            b0 = jnp.concatenate([r0_row, v_km2[:-1, :]], axis=0)
            b1 = jnp.concatenate([r1_row, v_km1[:-1, :]], axis=0)
            return finish(k, b0, b1, v_km1)

        def body_wrap(k, v_km2, v_km1):
            return finish(k, jnp.roll(v_km2, 1, axis=0),
                          jnp.roll(v_km1, 1, axis=0), v_km1)

        return body, body_wrap

    Q = N // 4

    def run(body_wrap, k_lo, k_hi, a, b, unroll=16):
        def step(k, carry):
            a, b = carry
            return (b, body_wrap(k, a, b))

        return jax.lax.fori_loop(k_lo, k_hi, step, (a, b), unroll=unroll)

    def shrink(body, k_first, a, b, cut):
        row_a = a[cut - 1:cut, :]
        row_b = b[cut - 1:cut, :]
        v0 = body(k_first, a[cut:, :], b[cut:, :], row_a, row_b)
        v1 = body(k_first + 1, b[cut:, :], v0, row_b, big_row)
        return v0, v1

    E = N // 8

    body_e0, wrap_e0 = make_body(x[:E, :], 0)
    big_e = big[:E, :]
    v2 = body_e0(2, big_e, big_e, zero_row, big_row)
    a, b = run(wrap_e0, 3, E + 2, big_e, v2)

    body_q0, wrap_q0 = make_body(x[:Q, :], 0)
    a, b = run(wrap_q0, E + 2, Q + 2,
               jnp.concatenate([a, big_e], axis=0),
               jnp.concatenate([b, big_e], axis=0))

    body_h0, wrap_h0 = make_body(x[:H, :], 0)
    big_q = big[:Q, :]
    a, b = run(wrap_h0, Q + 2, H + 2,
               jnp.concatenate([a, big_q], axis=0),
               jnp.concatenate([b, big_q], axis=0))

    body_full, wrap_full = make_body(x, 0)
    big_h = big[:H, :]
    a, b = run(wrap_full, H + 2, N + H + 1,
               jnp.concatenate([a, big_h], axis=0),
               jnp.concatenate([b, big_h], axis=0))

    body_hi, wrap_hi = make_body(x[H:, :], H)
    v0, v1 = shrink(body_hi, N + H + 1, a, b, H)
    a, b = run(wrap_hi, N + H + 3, N + M - Q + 1, v0, v1)

    body_q1, wrap_q1 = make_body(x[N - Q:, :], N - Q)
    v0, v1 = shrink(body_q1, N + M - Q + 1, a, b, Q)
    a, b = run(wrap_q1, N + M - Q + 3, N + M - E + 1, v0, v1)

    body_e1, wrap_e1 = make_body(x[N - E:, :], N - E)
    v0, v1 = shrink(body_e1, N + M - E + 1, a, b, Q - E)
    _, v_last = run(wrap_e1, N + M - E + 3, N + M + 1, v0, v1)
    out_ref[0, 0, :] = v_last[E - 1, :] * _C2


def kernel(x, y):
    B, N = x.shape
    M = y.shape[1]
    scale = jnp.float32(-_C1)
    x_t = (x * scale).T
    y_rev = (y * scale)[:, ::-1].T
    pad_left = N - 1
    total = pad_left + M + (N - 1)
    padded = ((total + 7) // 8) * 8
    y_pad = jnp.zeros((padded, B), jnp.float32).at[pad_left:pad_left + M].set(y_rev)

    Bb = 128
    NB = B // Bb
    out = pl.pallas_call(
        functools.partial(_sdtw_kernel, N=N, M=M),
        grid=(NB,),
        in_specs=[
            pl.BlockSpec((N, Bb), lambda i: (0, i)),
            pl.BlockSpec((padded, Bb), lambda i: (0, i)),
        ],
        out_specs=pl.BlockSpec((1, 1, Bb), lambda i: (i, 0, 0)),
        out_shape=jax.ShapeDtypeStruct((NB, 1, Bb), jnp.float32),
        compiler_params=pltpu.CompilerParams(dimension_semantics=("parallel",)),
    )(x_t, y_pad)
    loss = out.reshape(B) / (N + M)
    return loss.mean()


# speedup vs baseline: 1.0175x; 1.0175x over previous
import functools
import math

import jax
import jax.numpy as jnp
from jax.experimental import pallas as pl
from jax.experimental.pallas import tpu as pltpu

_GAMMA = 0.1
_BIG = 1e6
_C1 = -math.log2(math.e) / _GAMMA
_C2 = -_GAMMA * math.log(2.0)


def _sdtw_kernel(x_ref, y_ref, out_ref, *, N, M):
    x = x_ref[:, :]
    big = x * 0.0 + _BIG * _C1
    big_row = big[:1, :]
    zero_row = big_row * 0.0
    H = N // 2

    def make_body(xs, off):
        L = xs.shape[0]

        def body(k, v_km2, v_km1, r0_row, r1_row):
            yw = y_ref[pl.ds(off + N + M - k, L), :]
            d = jnp.abs(xs - yw)
            b0 = jnp.concatenate([r0_row, v_km2[:-1, :]], axis=0)
            b1 = jnp.concatenate([r1_row, v_km1[:-1, :]], axis=0)
            b2 = v_km1
            bmax = jnp.maximum(jnp.maximum(b0, b1), b2)
            rsum = jnp.exp2(b0 - bmax) + jnp.exp2(b1 - bmax) + jnp.exp2(b2 - bmax)
            return (jnp.log2(rsum) + bmax) - d

        return body

    Q = N // 4

    def run(body, k_lo, k_hi, a, b, unroll=32):
        def step(k, carry):
            a, b = carry
            return (b, body(k, a, b, big_row, big_row))

        return jax.lax.fori_loop(k_lo, k_hi, step, (a, b), unroll=unroll)

    def shrink(body, k_first, a, b, cut):
        row_a = a[cut - 1:cut, :]
        row_b = b[cut - 1:cut, :]
        v0 = body(k_first, a[cut:, :], b[cut:, :], row_a, row_b)
        v1 = body(k_first + 1, b[cut:, :], v0, row_b, big_row)
        return v0, v1

    E = N // 8

    body_e0 = make_body(x[:E, :], 0)
    big_e = big[:E, :]
    v2 = body_e0(2, big_e, big_e, zero_row, big_row)
    a, b = run(body_e0, 3, E + 2, big_e, v2)

    body_q0 = make_body(x[:Q, :], 0)
    a, b = run(body_q0, E + 2, Q + 2,
               jnp.concatenate([a, big_e], axis=0),
               jnp.concatenate([b, big_e], axis=0))

    body_h0 = make_body(x[:H, :], 0)
    big_q = big[:Q, :]
    a, b = run(body_h0, Q + 2, H + 2,
               jnp.concatenate([a, big_q], axis=0),
               jnp.concatenate([b, big_q], axis=0))

    body_full = make_body(x, 0)
    big_h = big[:H, :]
    a, b = run(body_full, H + 2, N + H + 1,
               jnp.concatenate([a, big_h], axis=0),
               jnp.concatenate([b, big_h], axis=0))

    body_hi = make_body(x[H:, :], H)
    v0, v1 = shrink(body_hi, N + H + 1, a, b, H)
    a, b = run(body_hi, N + H + 3, N + M - Q + 1, v0, v1)

    body_q1 = make_body(x[N - Q:, :], N - Q)
    v0, v1 = shrink(body_q1, N + M - Q + 1, a, b, Q)
    a, b = run(body_q1, N + M - Q + 3, N + M - E + 1, v0, v1)

    body_e1 = make_body(x[N - E:, :], N - E)
    v0, v1 = shrink(body_e1, N + M - E + 1, a, b, Q - E)
    _, v_last = run(body_e1, N + M - E + 3, N + M + 1, v0, v1)
    out_ref[0, 0, :] = v_last[E - 1, :] * _C2


def kernel(x, y):
    B, N = x.shape
    M = y.shape[1]
    scale = jnp.float32(-_C1)
    x_t = (x * scale).T
    y_rev = (y * scale)[:, ::-1].T
    pad_left = N - 1
    total = pad_left + M + (N - 1)
    padded = ((total + 7) // 8) * 8
    y_pad = jnp.zeros((padded, B), jnp.float32).at[pad_left:pad_left + M].set(y_rev)

    Bb = 128
    NB = B // Bb
    out = pl.pallas_call(
        functools.partial(_sdtw_kernel, N=N, M=M),
        grid=(NB,),
        in_specs=[
            pl.BlockSpec((N, Bb), lambda i: (0, i)),
            pl.BlockSpec((padded, Bb), lambda i: (0, i)),
        ],
        out_specs=pl.BlockSpec((1, 1, Bb), lambda i: (i, 0, 0)),
        out_shape=jax.ShapeDtypeStruct((NB, 1, Bb), jnp.float32),
        compiler_params=pltpu.CompilerParams(dimension_semantics=("parallel",)),
    )(x_t, y_pad)
    loss = out.reshape(B) / (N + M)
    return loss.mean()
